# baseline (speedup 1.0000x reference)
import functools

import jax
import jax.numpy as jnp
from jax import lax
from jax.experimental import pallas as pl
from jax.experimental.pallas import tpu as pltpu
from jax.experimental.pallas import tpu_sc as plsc

D = 128
NC = 2
NS = 16
CHUNK = 128

_MESH = plsc.VectorSubcoreMesh(
    core_axis_name="c", subcore_axis_name="s", num_cores=NC, num_subcores=NS
)

_F_CORE0 = 0.25


@functools.lru_cache(maxsize=None)
def _make_deg_kernel(chunks_per_tile: int, n_pad: int):
    rows_per_tile = n_pad // NS

    @functools.partial(
        pl.kernel,
        out_type=jax.ShapeDtypeStruct((NC, n_pad, D), jnp.float32),
        mesh=_MESH,
        scratch_types=[
            pltpu.VMEM((8, CHUNK), jnp.int32),
            pltpu.VMEM((8, CHUNK), jnp.int32),
            pltpu.VMEM((CHUNK, D), jnp.float32),
            pltpu.VMEM((CHUNK, D), jnp.float32),
            pltpu.VMEM_SHARED((n_pad, D), jnp.float32),
            pltpu.SemaphoreType.DMA,
        ],
    )
    def deg_kernel(src_hbm, dst_hbm, e0_hbm, e1_hbm, zeros_hbm, out_hbm,
                   src_v, dst_v, e0_v, e1_v, acc, ssem):
        c = lax.axis_index("c")
        s = lax.axis_index("s")
        base = (c * NS + s) * chunks_per_tile
        pltpu.sync_copy(e0_hbm, e0_v)
        pltpu.sync_copy(e1_hbm, e1_v)
        r0 = s * rows_per_tile
        pltpu.sync_copy(zeros_hbm.at[pl.ds(r0, rows_per_tile)],
                        acc.at[pl.ds(r0, rows_per_tile)])
        plsc.subcore_barrier()

        @pl.loop(0, chunks_per_tile // 8)
        def _(g):
            pltpu.sync_copy(src_hbm.at[pl.ds(base + g * 8, 8)], src_v)
            pltpu.sync_copy(dst_hbm.at[pl.ds(base + g * 8, 8)], dst_v)
            descs = []
            for j in range(8):
                descs.append(
                    pltpu.async_copy(e0_v, acc.at[src_v.at[j]], ssem, add=True))
                descs.append(
                    pltpu.async_copy(e1_v, acc.at[dst_v.at[j]], ssem, add=True))
            for d in descs:
                d.wait()

        plsc.subcore_barrier()
        pltpu.sync_copy(acc.at[pl.ds(r0, rows_per_tile)],
                        out_hbm.at[c, pl.ds(r0, rows_per_tile)])

    return deg_kernel


@functools.lru_cache(maxsize=None)
def _make_edge_kernel(cpt0: int, cpt1: int, n_pad: int):
    rows_per_tile = n_pad // NS
    group = 8
    max_cpt = max(cpt0, cpt1)

    @functools.partial(
        pl.kernel,
        out_type=jax.ShapeDtypeStruct((NC, n_pad, D), jnp.float32),
        mesh=_MESH,
        scratch_types=[
            pltpu.VMEM((group, CHUNK), jnp.int32),
            pltpu.VMEM((max_cpt, CHUNK), jnp.int32),
            pltpu.VMEM((CHUNK, D), jnp.float32),
            pltpu.VMEM((CHUNK, D), jnp.float32),
            pltpu.VMEM_SHARED((n_pad, D), jnp.float32),
            pltpu.SemaphoreType.DMA,
            pltpu.SemaphoreType.DMA,
            pltpu.SemaphoreType.DMA,
            pltpu.SemaphoreType.DMA,
        ],
    )
    def edge_kernel(hws_hbm, src_hbm, dst_hbm, zeros_hbm, out_hbm,
                    sidx, didx, rows_a, rows_b, acc, ga, gb, sa, sb):
        c = lax.axis_index("c")
        s = lax.axis_index("s")
        base = jnp.where(c == 0, s * cpt0, NS * cpt0 + s * cpt1)
        n_groups = jnp.where(c == 0, cpt0 // group, cpt1 // group)
        pltpu.sync_copy(dst_hbm.at[pl.ds(base, max_cpt)], didx)
        r0 = s * rows_per_tile
        pltpu.sync_copy(zeros_hbm.at[pl.ds(r0, rows_per_tile)],
                        acc.at[pl.ds(r0, rows_per_tile)])
        plsc.subcore_barrier()

        bufs = (rows_a, rows_b)
        gsems = (ga, gb)
        ssems = (sa, sb)

        @pl.loop(0, n_groups)
        def _(g):
            pltpu.sync_copy(src_hbm.at[pl.ds(base + g * group, group)], sidx)
            gd = [None, None]
            sd = [None, None]
            gd[0] = pltpu.async_copy(hws_hbm.at[sidx.at[0]], rows_a, ga)
            for j in range(group):
                p = j & 1
                o = 1 - p
                gd[p].wait()
                sd[p] = pltpu.async_copy(
                    bufs[p], acc.at[didx.at[g * group + j]], ssems[p], add=True)
                if j + 1 < group:
                    if sd[o] is not None:
                        sd[o].wait()
                    gd[o] = pltpu.async_copy(
                        hws_hbm.at[sidx.at[j + 1]], bufs[o], gsems[o])
            sd[0].wait()
            sd[1].wait()

        plsc.subcore_barrier()
        pltpu.sync_copy(acc.at[pl.ds(r0, rows_per_tile)],
                        out_hbm.at[c, pl.ds(r0, rows_per_tile)])

    return edge_kernel


def _norm_col(parts, which):
    deg = parts[0] + parts[1]
    d = deg[:, which:which + 1]
    return jnp.where(d > 0, lax.rsqrt(jnp.maximum(d, 1.0)), 0.0)


def _stage_first_body(x_ref, w_ref, parts_ref, o_ref):
    ns = _norm_col(parts_ref[...], 0)
    xw = jnp.dot(x_ref[...], w_ref[...], preferred_element_type=jnp.float32,
                 precision=lax.Precision.HIGHEST)
    o_ref[...] = xw * ns


def _stage_mid_body(p_ref, parts_ref, b_ref, w_ref, o_ref):
    parts = parts_ref[...]
    nd = _norm_col(parts, 1)
    ns = _norm_col(parts, 0)
    agg = p_ref[0] + p_ref[1]
    h = jnp.maximum(agg * nd + b_ref[...], 0.0)
    hw = jnp.dot(h, w_ref[...], preferred_element_type=jnp.float32,
                 precision=lax.Precision.HIGHEST)
    o_ref[...] = hw * ns


def _stage_final_body(p_ref, parts_ref, b_ref, o_ref):
    nd = _norm_col(parts_ref[...], 1)
    agg = p_ref[0] + p_ref[1]
    o_ref[...] = agg * nd + b_ref[...]


@functools.lru_cache(maxsize=None)
def _make_tc_kernels(n_pad: int):
    f32 = jnp.float32
    grid = 8
    blk = n_pad // grid
    out = jax.ShapeDtypeStruct((n_pad, D), f32)
    x_spec = pl.BlockSpec((blk, D), lambda i: (i, 0))
    p_spec = pl.BlockSpec((2, blk, D), lambda i: (0, i, 0))
    parts_spec = pl.BlockSpec((2, blk, D), lambda i: (0, i, 0))
    w_spec = pl.BlockSpec((D, D), lambda i: (0, 0))
    b_spec = pl.BlockSpec((1, D), lambda i: (0, 0))
    o_spec = pl.BlockSpec((blk, D), lambda i: (i, 0))
    first = pl.pallas_call(
        _stage_first_body, grid=(grid,), out_shape=out,
        in_specs=[x_spec, w_spec, parts_spec], out_specs=o_spec)
    mid = pl.pallas_call(
        _stage_mid_body, grid=(grid,), out_shape=out,
        in_specs=[p_spec, parts_spec, b_spec, w_spec], out_specs=o_spec)
    final = pl.pallas_call(
        _stage_final_body, grid=(grid,), out_shape=out,
        in_specs=[p_spec, parts_spec, b_spec], out_specs=o_spec)
    return first, mid, final


def kernel(x, edge_index, W1, b1, W2, b2, W3, b3):
    n = x.shape[0]
    e = edge_index.shape[1]
    n_classes = W3.shape[1]
    tiles = NC * NS
    total_cpt = -(-e // (NS * CHUNK * 16)) * 16
    e_pad = NS * total_cpt * CHUNK
    cpt0 = (int(total_cpt * _F_CORE0) // 8) * 8
    cpt1 = total_cpt - cpt0
    extra_rows = max(0, cpt0 - cpt1)
    n_pad = -(-(n + 1) // (NS * 8)) * NS * 8

    src = edge_index[0].astype(jnp.int32)
    dst = edge_index[1].astype(jnp.int32)
    fill = jnp.full((e_pad - e + extra_rows * CHUNK,), n, dtype=jnp.int32)
    src2d = jnp.concatenate([src, fill]).reshape(-1, CHUNK)
    dst2d = jnp.concatenate([dst, fill]).reshape(-1, CHUNK)

    x_p = jnp.zeros((n_pad, D), jnp.float32).at[:n, : x.shape[1]].set(x)
    zeros_d = jnp.zeros((n_pad, D), jnp.float32)
    lane = lax.broadcasted_iota(jnp.int32, (CHUNK, D), 1)
    e0 = (lane == 0).astype(jnp.float32)
    e1 = (lane == 1).astype(jnp.float32)
    W3p = jnp.zeros((D, D), jnp.float32).at[: W3.shape[0], :n_classes].set(W3)
    b3p = jnp.zeros((D,), jnp.float32).at[:n_classes].set(b3)

    deg = _make_deg_kernel(total_cpt // 2, n_pad)(src2d, dst2d, e0, e1, zeros_d)
    edge = _make_edge_kernel(cpt0, cpt1, n_pad)
    first, mid, final = _make_tc_kernels(n_pad)

    hws1 = first(x_p, W1, deg)
    p1 = edge(hws1, src2d, dst2d, zeros_d)
    hws2 = mid(p1, deg, b1.reshape(1, D), W2)
    p2 = edge(hws2, src2d, dst2d, zeros_d)
    hws3 = mid(p2, deg, b2.reshape(1, D), W3p)
    p3 = edge(hws3, src2d, dst2d, zeros_d)
    out = final(p3, deg, b3p.reshape(1, D))
    return out[:n, :n_classes]


# speedup vs baseline: 1.4017x; 1.4017x over previous
import functools

import jax
import jax.numpy as jnp
from jax import lax
from jax.experimental import pallas as pl
from jax.experimental.pallas import tpu as pltpu
from jax.experimental.pallas import tpu_sc as plsc

D = 128
NC = 2
NS = 16
CHUNK = 128

_MESH = plsc.VectorSubcoreMesh(
    core_axis_name="c", subcore_axis_name="s", num_cores=NC, num_subcores=NS
)

_F_CORE0 = 0.8


@functools.lru_cache(maxsize=None)
def _make_deg_kernel(chunks_per_tile: int, n_pad: int):
    rows_per_tile = n_pad // NS

    @functools.partial(
        pl.kernel,
        out_type=jax.ShapeDtypeStruct((NC, n_pad, D), jnp.float32),
        mesh=_MESH,
        scratch_types=[
            pltpu.VMEM((8, CHUNK), jnp.int32),
            pltpu.VMEM((8, CHUNK), jnp.int32),
            pltpu.VMEM((CHUNK, D), jnp.float32),
            pltpu.VMEM((CHUNK, D), jnp.float32),
            pltpu.VMEM_SHARED((n_pad, D), jnp.float32),
            pltpu.SemaphoreType.DMA,
        ],
    )
    def deg_kernel(src_hbm, dst_hbm, e0_hbm, e1_hbm, zeros_hbm, out_hbm,
                   src_v, dst_v, e0_v, e1_v, acc, ssem):
        c = lax.axis_index("c")
        s = lax.axis_index("s")
        base = (c * NS + s) * chunks_per_tile
        pltpu.sync_copy(e0_hbm, e0_v)
        pltpu.sync_copy(e1_hbm, e1_v)
        r0 = s * rows_per_tile
        pltpu.sync_copy(zeros_hbm.at[pl.ds(r0, rows_per_tile)],
                        acc.at[pl.ds(r0, rows_per_tile)])
        plsc.subcore_barrier()

        @pl.loop(0, chunks_per_tile // 8)
        def _(g):
            pltpu.sync_copy(src_hbm.at[pl.ds(base + g * 8, 8)], src_v)
            pltpu.sync_copy(dst_hbm.at[pl.ds(base + g * 8, 8)], dst_v)
            descs = []
            for j in range(8):
                descs.append(
                    pltpu.async_copy(e0_v, acc.at[src_v.at[j]], ssem, add=True))
                descs.append(
                    pltpu.async_copy(e1_v, acc.at[dst_v.at[j]], ssem, add=True))
            for d in descs:
                d.wait()

        plsc.subcore_barrier()
        pltpu.sync_copy(acc.at[pl.ds(r0, rows_per_tile)],
                        out_hbm.at[c, pl.ds(r0, rows_per_tile)])

    return deg_kernel


@functools.lru_cache(maxsize=None)
def _make_edge_kernel(cpt0: int, cpt1: int, n_pad: int):
    rows_per_tile = n_pad // NS
    group = 8
    max_cpt = max(cpt0, cpt1)

    @functools.partial(
        pl.kernel,
        out_type=jax.ShapeDtypeStruct((NC, n_pad, D), jnp.float32),
        mesh=_MESH,
        scratch_types=[
            pltpu.VMEM((group, CHUNK), jnp.int32),
            pltpu.VMEM((max_cpt, CHUNK), jnp.int32),
            pltpu.VMEM((CHUNK, D), jnp.float32),
            pltpu.VMEM((CHUNK, D), jnp.float32),
            pltpu.VMEM_SHARED((n_pad, D), jnp.float32),
            pltpu.SemaphoreType.DMA,
            pltpu.SemaphoreType.DMA,
            pltpu.SemaphoreType.DMA,
            pltpu.SemaphoreType.DMA,
        ],
    )
    def edge_kernel(hws_hbm, src_hbm, dst_hbm, zeros_hbm, out_hbm,
                    sidx, didx, rows_a, rows_b, acc, ga, gb, sa, sb):
        c = lax.axis_index("c")
        s = lax.axis_index("s")
        base = jnp.where(c == 0, s * cpt0, NS * cpt0 + s * cpt1)
        n_groups = jnp.where(c == 0, cpt0 // group, cpt1 // group)
        pltpu.sync_copy(dst_hbm.at[pl.ds(base, max_cpt)], didx)
        r0 = s * rows_per_tile
        pltpu.sync_copy(zeros_hbm.at[pl.ds(r0, rows_per_tile)],
                        acc.at[pl.ds(r0, rows_per_tile)])
        plsc.subcore_barrier()

        bufs = (rows_a, rows_b)
        gsems = (ga, gb)
        ssems = (sa, sb)

        @pl.loop(0, n_groups)
        def _(g):
            pltpu.sync_copy(src_hbm.at[pl.ds(base + g * group, group)], sidx)
            gd = [None, None]
            sd = [None, None]
            gd[0] = pltpu.async_copy(hws_hbm.at[sidx.at[0]], rows_a, ga)
            for j in range(group):
                p = j & 1
                o = 1 - p
                gd[p].wait()
                sd[p] = pltpu.async_copy(
                    bufs[p], acc.at[didx.at[g * group + j]], ssems[p], add=True)
                if j + 1 < group:
                    if sd[o] is not None:
                        sd[o].wait()
                    gd[o] = pltpu.async_copy(
                        hws_hbm.at[sidx.at[j + 1]], bufs[o], gsems[o])
            sd[0].wait()
            sd[1].wait()

        plsc.subcore_barrier()
        pltpu.sync_copy(acc.at[pl.ds(r0, rows_per_tile)],
                        out_hbm.at[c, pl.ds(r0, rows_per_tile)])

    return edge_kernel


def _norm_col(parts, which):
    deg = parts[0] + parts[1]
    d = deg[:, which:which + 1]
    return jnp.where(d > 0, lax.rsqrt(jnp.maximum(d, 1.0)), 0.0)


def _stage_first_body(x_ref, w_ref, parts_ref, o_ref):
    ns = _norm_col(parts_ref[...], 0)
    xw = jnp.dot(x_ref[...], w_ref[...], preferred_element_type=jnp.float32,
                 precision=lax.Precision.HIGHEST)
    o_ref[...] = xw * ns


def _stage_mid_body(p_ref, parts_ref, b_ref, w_ref, o_ref):
    parts = parts_ref[...]
    nd = _norm_col(parts, 1)
    ns = _norm_col(parts, 0)
    agg = p_ref[0] + p_ref[1]
    h = jnp.maximum(agg * nd + b_ref[...], 0.0)
    hw = jnp.dot(h, w_ref[...], preferred_element_type=jnp.float32,
                 precision=lax.Precision.HIGHEST)
    o_ref[...] = hw * ns


def _stage_final_body(p_ref, parts_ref, b_ref, o_ref):
    nd = _norm_col(parts_ref[...], 1)
    agg = p_ref[0] + p_ref[1]
    o_ref[...] = agg * nd + b_ref[...]


@functools.lru_cache(maxsize=None)
def _make_tc_kernels(n_pad: int):
    f32 = jnp.float32
    grid = 8
    blk = n_pad // grid
    out = jax.ShapeDtypeStruct((n_pad, D), f32)
    x_spec = pl.BlockSpec((blk, D), lambda i: (i, 0))
    p_spec = pl.BlockSpec((2, blk, D), lambda i: (0, i, 0))
    parts_spec = pl.BlockSpec((2, blk, D), lambda i: (0, i, 0))
    w_spec = pl.BlockSpec((D, D), lambda i: (0, 0))
    b_spec = pl.BlockSpec((1, D), lambda i: (0, 0))
    o_spec = pl.BlockSpec((blk, D), lambda i: (i, 0))
    first = pl.pallas_call(
        _stage_first_body, grid=(grid,), out_shape=out,
        in_specs=[x_spec, w_spec, parts_spec], out_specs=o_spec)
    mid = pl.pallas_call(
        _stage_mid_body, grid=(grid,), out_shape=out,
        in_specs=[p_spec, parts_spec, b_spec, w_spec], out_specs=o_spec)
    final = pl.pallas_call(
        _stage_final_body, grid=(grid,), out_shape=out,
        in_specs=[p_spec, parts_spec, b_spec], out_specs=o_spec)
    return first, mid, final


def kernel(x, edge_index, W1, b1, W2, b2, W3, b3):
    n = x.shape[0]
    e = edge_index.shape[1]
    n_classes = W3.shape[1]
    tiles = NC * NS
    total_cpt = -(-e // (NS * CHUNK * 16)) * 16
    e_pad = NS * total_cpt * CHUNK
    cpt0 = (int(total_cpt * _F_CORE0) // 8) * 8
    cpt1 = total_cpt - cpt0
    extra_rows = max(0, cpt0 - cpt1)
    n_pad = -(-(n + 1) // (NS * 8)) * NS * 8

    src = edge_index[0].astype(jnp.int32)
    dst = edge_index[1].astype(jnp.int32)
    fill = jnp.full((e_pad - e + extra_rows * CHUNK,), n, dtype=jnp.int32)
    src2d = jnp.concatenate([src, fill]).reshape(-1, CHUNK)
    dst2d = jnp.concatenate([dst, fill]).reshape(-1, CHUNK)

    x_p = jnp.zeros((n_pad, D), jnp.float32).at[:n, : x.shape[1]].set(x)
    zeros_d = jnp.zeros((n_pad, D), jnp.float32)
    lane = lax.broadcasted_iota(jnp.int32, (CHUNK, D), 1)
    e0 = (lane == 0).astype(jnp.float32)
    e1 = (lane == 1).astype(jnp.float32)
    W3p = jnp.zeros((D, D), jnp.float32).at[: W3.shape[0], :n_classes].set(W3)
    b3p = jnp.zeros((D,), jnp.float32).at[:n_classes].set(b3)

    deg = _make_deg_kernel(total_cpt // 2, n_pad)(src2d, dst2d, e0, e1, zeros_d)
    edge = _make_edge_kernel(cpt0, cpt1, n_pad)
    first, mid, final = _make_tc_kernels(n_pad)

    hws1 = first(x_p, W1, deg)
    p1 = edge(hws1, src2d, dst2d, zeros_d)
    hws2 = mid(p1, deg, b1.reshape(1, D), W2)
    p2 = edge(hws2, src2d, dst2d, zeros_d)
    hws3 = mid(p2, deg, b2.reshape(1, D), W3p)
    p3 = edge(hws3, src2d, dst2d, zeros_d)
    out = final(p3, deg, b3p.reshape(1, D))
    return out[:n, :n_classes]


# speedup vs baseline: 1.5888x; 1.1335x over previous
import functools

import jax
import jax.numpy as jnp
from jax import lax
from jax.experimental import pallas as pl
from jax.experimental.pallas import tpu as pltpu
from jax.experimental.pallas import tpu_sc as plsc

D = 128
NC = 2
NS = 16
CHUNK = 128

_MESH = plsc.VectorSubcoreMesh(
    core_axis_name="c", subcore_axis_name="s", num_cores=NC, num_subcores=NS
)

_F_CORE0 = 0.85


@functools.lru_cache(maxsize=None)
def _make_deg_kernel(chunks_per_tile: int, n_pad: int):
    rows_per_tile = n_pad // NS

    @functools.partial(
        pl.kernel,
        out_type=jax.ShapeDtypeStruct((NC, n_pad, D), jnp.float32),
        mesh=_MESH,
        scratch_types=[
            pltpu.VMEM((8, CHUNK), jnp.int32),
            pltpu.VMEM((8, CHUNK), jnp.int32),
            pltpu.VMEM((CHUNK, D), jnp.float32),
            pltpu.VMEM((CHUNK, D), jnp.float32),
            pltpu.VMEM_SHARED((n_pad, D), jnp.float32),
            pltpu.SemaphoreType.DMA,
        ],
    )
    def deg_kernel(src_hbm, dst_hbm, e0_hbm, e1_hbm, zeros_hbm, out_hbm,
                   src_v, dst_v, e0_v, e1_v, acc, ssem):
        c = lax.axis_index("c")
        s = lax.axis_index("s")
        base = (c * NS + s) * chunks_per_tile
        pltpu.sync_copy(e0_hbm, e0_v)
        pltpu.sync_copy(e1_hbm, e1_v)
        r0 = s * rows_per_tile
        pltpu.sync_copy(zeros_hbm.at[pl.ds(r0, rows_per_tile)],
                        acc.at[pl.ds(r0, rows_per_tile)])
        plsc.subcore_barrier()

        @pl.loop(0, chunks_per_tile // 8)
        def _(g):
            pltpu.sync_copy(src_hbm.at[pl.ds(base + g * 8, 8)], src_v)
            pltpu.sync_copy(dst_hbm.at[pl.ds(base + g * 8, 8)], dst_v)
            descs = []
            for j in range(8):
                descs.append(
                    pltpu.async_copy(e0_v, acc.at[src_v.at[j]], ssem, add=True))
                descs.append(
                    pltpu.async_copy(e1_v, acc.at[dst_v.at[j]], ssem, add=True))
            for d in descs:
                d.wait()

        plsc.subcore_barrier()
        pltpu.sync_copy(acc.at[pl.ds(r0, rows_per_tile)],
                        out_hbm.at[c, pl.ds(r0, rows_per_tile)])

    return deg_kernel


@functools.lru_cache(maxsize=None)
def _make_edge_kernel(cpt0: int, cpt1: int, n_pad: int):
    rows_per_tile = n_pad // NS
    group = 8

    @functools.partial(
        pl.kernel,
        out_type=jax.ShapeDtypeStruct((NC, n_pad, D), jnp.float32),
        mesh=_MESH,
        scratch_types=[
            pltpu.VMEM((group, CHUNK), jnp.int32),
            pltpu.VMEM((group, CHUNK), jnp.int32),
            pltpu.VMEM((CHUNK, D), jnp.float32),
            pltpu.VMEM((CHUNK, D), jnp.float32),
            pltpu.VMEM_SHARED((n_pad, D), jnp.float32),
            pltpu.SemaphoreType.DMA,
            pltpu.SemaphoreType.DMA,
            pltpu.SemaphoreType.DMA,
            pltpu.SemaphoreType.DMA,
        ],
    )
    def edge_kernel(hws_hbm, src_hbm, dst_hbm, zeros_hbm, out_hbm,
                    sidx, didx, rows_a, rows_b, acc, ga, gb, sa, sb):
        c = lax.axis_index("c")
        s = lax.axis_index("s")
        base = jnp.where(c == 0, s * cpt0, NS * cpt0 + s * cpt1)
        n_groups = jnp.where(c == 0, cpt0 // group, cpt1 // group)
        r0 = s * rows_per_tile
        pltpu.sync_copy(zeros_hbm.at[pl.ds(r0, rows_per_tile)],
                        acc.at[pl.ds(r0, rows_per_tile)])
        plsc.subcore_barrier()

        bufs = (rows_a, rows_b)
        gsems = (ga, gb)
        ssems = (sa, sb)

        @pl.loop(0, n_groups)
        def _(g):
            pltpu.sync_copy(src_hbm.at[pl.ds(base + g * group, group)], sidx)
            pltpu.sync_copy(dst_hbm.at[pl.ds(base + g * group, group)], didx)
            gd = [None, None]
            sd = [None, None]
            gd[0] = pltpu.async_copy(hws_hbm.at[sidx.at[0]], rows_a, ga)
            for j in range(group):
                p = j & 1
                o = 1 - p
                gd[p].wait()
                sd[p] = pltpu.async_copy(
                    bufs[p], acc.at[didx.at[j]], ssems[p], add=True)
                if j + 1 < group:
                    if sd[o] is not None:
                        sd[o].wait()
                    gd[o] = pltpu.async_copy(
                        hws_hbm.at[sidx.at[j + 1]], bufs[o], gsems[o])
            sd[0].wait()
            sd[1].wait()

        plsc.subcore_barrier()
        pltpu.sync_copy(acc.at[pl.ds(r0, rows_per_tile)],
                        out_hbm.at[c, pl.ds(r0, rows_per_tile)])

    return edge_kernel


def _norm_col(parts, which):
    deg = parts[0] + parts[1]
    d = deg[:, which:which + 1]
    return jnp.where(d > 0, lax.rsqrt(jnp.maximum(d, 1.0)), 0.0)


def _stage_first_body(x_ref, w_ref, parts_ref, o_ref):
    ns = _norm_col(parts_ref[...], 0)
    xw = jnp.dot(x_ref[...], w_ref[...], preferred_element_type=jnp.float32,
                 precision=lax.Precision.HIGHEST)
    o_ref[...] = xw * ns


def _stage_mid_body(p_ref, parts_ref, b_ref, w_ref, o_ref):
    parts = parts_ref[...]
    nd = _norm_col(parts, 1)
    ns = _norm_col(parts, 0)
    agg = p_ref[0] + p_ref[1]
    h = jnp.maximum(agg * nd + b_ref[...], 0.0)
    hw = jnp.dot(h, w_ref[...], preferred_element_type=jnp.float32,
                 precision=lax.Precision.HIGHEST)
    o_ref[...] = hw * ns


def _stage_final_body(p_ref, parts_ref, b_ref, o_ref):
    nd = _norm_col(parts_ref[...], 1)
    agg = p_ref[0] + p_ref[1]
    o_ref[...] = agg * nd + b_ref[...]


@functools.lru_cache(maxsize=None)
def _make_tc_kernels(n_pad: int):
    f32 = jnp.float32
    grid = 8
    blk = n_pad // grid
    out = jax.ShapeDtypeStruct((n_pad, D), f32)
    x_spec = pl.BlockSpec((blk, D), lambda i: (i, 0))
    p_spec = pl.BlockSpec((2, blk, D), lambda i: (0, i, 0))
    parts_spec = pl.BlockSpec((2, blk, D), lambda i: (0, i, 0))
    w_spec = pl.BlockSpec((D, D), lambda i: (0, 0))
    b_spec = pl.BlockSpec((1, D), lambda i: (0, 0))
    o_spec = pl.BlockSpec((blk, D), lambda i: (i, 0))
    first = pl.pallas_call(
        _stage_first_body, grid=(grid,), out_shape=out,
        in_specs=[x_spec, w_spec, parts_spec], out_specs=o_spec)
    mid = pl.pallas_call(
        _stage_mid_body, grid=(grid,), out_shape=out,
        in_specs=[p_spec, parts_spec, b_spec, w_spec], out_specs=o_spec)
    final = pl.pallas_call(
        _stage_final_body, grid=(grid,), out_shape=out,
        in_specs=[p_spec, parts_spec, b_spec], out_specs=o_spec)
    return first, mid, final


def kernel(x, edge_index, W1, b1, W2, b2, W3, b3):
    n = x.shape[0]
    e = edge_index.shape[1]
    n_classes = W3.shape[1]
    tiles = NC * NS
    total_cpt = -(-e // (NS * CHUNK * 16)) * 16
    e_pad = NS * total_cpt * CHUNK
    cpt0 = (int(total_cpt * _F_CORE0) // 8) * 8
    cpt1 = total_cpt - cpt0
    n_pad = -(-(n + 1) // (NS * 8)) * NS * 8

    src = edge_index[0].astype(jnp.int32)
    dst = edge_index[1].astype(jnp.int32)
    fill = jnp.full((e_pad - e,), n, dtype=jnp.int32)
    src2d = jnp.concatenate([src, fill]).reshape(-1, CHUNK)
    dst2d = jnp.concatenate([dst, fill]).reshape(-1, CHUNK)

    x_p = jnp.zeros((n_pad, D), jnp.float32).at[:n, : x.shape[1]].set(x)
    zeros_d = jnp.zeros((n_pad, D), jnp.float32)
    lane = lax.broadcasted_iota(jnp.int32, (CHUNK, D), 1)
    e0 = (lane == 0).astype(jnp.float32)
    e1 = (lane == 1).astype(jnp.float32)
    W3p = jnp.zeros((D, D), jnp.float32).at[: W3.shape[0], :n_classes].set(W3)
    b3p = jnp.zeros((D,), jnp.float32).at[:n_classes].set(b3)

    deg = _make_deg_kernel(total_cpt // 2, n_pad)(src2d, dst2d, e0, e1, zeros_d)
    edge = _make_edge_kernel(cpt0, cpt1, n_pad)
    first, mid, final = _make_tc_kernels(n_pad)

    hws1 = first(x_p, W1, deg)
    p1 = edge(hws1, src2d, dst2d, zeros_d)
    hws2 = mid(p1, deg, b1.reshape(1, D), W2)
    p2 = edge(hws2, src2d, dst2d, zeros_d)
    hws3 = mid(p2, deg, b2.reshape(1, D), W3p)
    p3 = edge(hws3, src2d, dst2d, zeros_d)
    out = final(p3, deg, b3p.reshape(1, D))
    return out[:n, :n_classes]
